# baseline (speedup 1.0000x reference)
import functools

import jax
import jax.numpy as jnp
from jax import lax
from jax.experimental import pallas as pl
from jax.experimental.pallas import tpu as pltpu
from jax.experimental.pallas import tpu_sc as plsc

F32 = jnp.float32
_RSQRT_MAGIC = 0x5F3759DF


def _silu(x):
    return x * jax.nn.sigmoid(x)


def _ln(x, g, b):
    m = jnp.mean(x, axis=-1, keepdims=True)
    v = jnp.var(x, axis=-1, keepdims=True)
    return (x - m) * lax.rsqrt(v + 1e-5) * g + b


def _mm(a, b):
    return jnp.dot(a, b, preferred_element_type=F32)


def _in_body(h, wi1, bi1, wi2, bi2, ng, nb, w1r, w1c, xn_o, a_o, b_o):
    x = _silu(_mm(h[...], wi1[...]) + bi1[...])
    x = _mm(x, wi2[...]) + bi2[...]
    xn = _ln(x, ng[...], nb[...])
    xn_o[...] = xn
    a_o[...] = _mm(xn, w1r[...]).astype(a_o.dtype)
    b_o[...] = _mm(xn, w1c[...]).astype(b_o.dtype)


def _mid_body(xn, s2, deg, ew2, eb2, nw1a, nw1b, nb1, nlg, nlb, nw2, nb2,
              ng, nb, w1r, w1c, xn_o, a_o, b_o):
    xn_v = xn[...]
    s = s2[0] + s2[1]
    degc = deg[0, :, 0:1] + deg[1, :, 0:1]
    agg = _mm(s, ew2[...]) + degc * eb2[...]
    a = _mm(xn_v, nw1a[...]) + _mm(agg, nw1b[...]) + nb1[...]
    a = _silu(_ln(a, nlg[...], nlb[...]))
    a = _mm(a, nw2[...]) + nb2[...]
    x_new = xn_v + a
    xn2 = _ln(x_new, ng[...], nb[...])
    xn_o[...] = xn2
    a_o[...] = _mm(xn2, w1r[...]).astype(a_o.dtype)
    b_o[...] = _mm(xn2, w1c[...]).astype(b_o.dtype)


def _fin_body(xn, s2, deg, ew2, eb2, nw1a, nw1b, nb1, nlg, nlb, nw2, nb2,
              ow1, ob1, ow2, ob2, out_o):
    xn_v = xn[...]
    s = s2[0] + s2[1]
    degc = deg[0, :, 0:1] + deg[1, :, 0:1]
    agg = _mm(s, ew2[...]) + degc * eb2[...]
    a = _mm(xn_v, nw1a[...]) + _mm(agg, nw1b[...]) + nb1[...]
    a = _silu(_ln(a, nlg[...], nlb[...]))
    a = _mm(a, nw2[...]) + nb2[...]
    x_new = xn_v + a
    o = _silu(_mm(x_new, ow1[...]) + ob1[...])
    out_o[...] = _mm(o, ow2[...]) + ob2[...]


def _c_body(ea, w1e_ev, b1_ev, w1e_od, b1_od, c_o):
    nl = w1e_ev.shape[0]
    ea_v = ea[...]
    for l in range(nl):
        cev = (_mm(ea_v, w1e_ev[l]) + b1_ev[l]).astype(jnp.bfloat16)
        cod = (_mm(ea_v, w1e_od[l]) + b1_od[l]).astype(jnp.bfloat16)
        lo = lax.convert_element_type(
            lax.bitcast_convert_type(cev, jnp.uint16), jnp.uint32)
        hi = lax.convert_element_type(
            lax.bitcast_convert_type(cod, jnp.uint16), jnp.uint32)
        c_o[l] = lax.bitcast_convert_type(lo | (hi << 16), jnp.int32)


def _full_spec(x):
    r = x.ndim
    return pl.BlockSpec(x.shape, lambda i, _r=r: (0,) * _r)


def _row_spec(bn, shape):
    rest = shape[1:]
    if len(shape) == 3:
        return pl.BlockSpec((shape[0], bn) + shape[2:], lambda i: (0, i, 0))
    return pl.BlockSpec((bn,) + rest, lambda i: (i,) + (0,) * len(rest))


def _tc_call(body, n_rows, bn, row_in_idx, ins, outs):
    grid = (n_rows // bn,)
    in_specs = []
    for i, x in enumerate(ins):
        in_specs.append(_row_spec(bn, x.shape) if i in row_in_idx else _full_spec(x))
    out_specs = [_row_spec(bn, o.shape) for o in outs]
    out_shape = [jax.ShapeDtypeStruct(o.shape, o.dtype) for o in outs]
    return pl.pallas_call(
        body,
        grid=grid,
        in_specs=in_specs,
        out_specs=out_specs if len(outs) > 1 else out_specs[0],
        out_shape=out_shape if len(outs) > 1 else out_shape[0],
    )(*ins)


def _edge_compute(buf_a, buf_b, buf_c, buf_u, lglb_v, scale_v, n_edges, hid):
    nv = hid // 16
    lg = [lglb_v[0, pl.ds(j * 16, 16)] for j in range(nv)]
    lb = [lglb_v[1, pl.ds(j * 16, 16)] for j in range(nv)]

    @plsc.parallel_loop(0, n_edges, 1, unroll=4)
    def pass1(e):
        sa = None
        qa = None
        for j in range(hid // 32):
            cw = plsc.bitcast(buf_c[e, pl.ds(j * 16, 16)], jnp.bfloat16)
            c0, c1 = plsc.unpack(cw, format=plsc.PackFormat.INTERLEAVED,
                                 preferred_element_type=F32)
            sl0 = pl.ds(j * 32, 16)
            sl1 = pl.ds(j * 32 + 16, 16)
            v0 = buf_a[e, sl0] + buf_b[e, sl0] + c0
            v1 = buf_a[e, sl1] + buf_b[e, sl1] + c1
            buf_u[e, sl0] = v0
            buf_u[e, sl1] = v1
            p = v0 + v1
            q = v0 * v0 + v1 * v1
            sa = p if sa is None else sa + p
            qa = q if qa is None else qa + q

        s1 = jnp.sum(sa)
        s2 = jnp.sum(qa)
        mean = s1 * (1.0 / hid)
        var = s2 * (1.0 / hid) - mean * mean + 1e-5
        hv = 0.5 * var
        bits = lax.bitcast_convert_type(var, jnp.int32)
        y = lax.bitcast_convert_type(jnp.int32(_RSQRT_MAGIC) - (bits >> 1),
                                     F32)
        for _ in range(2):
            y = y * (1.5 - hv * y * y)
        scale_v[e, pl.ds(0, 16)] = jnp.full((16,), y, F32)
        scale_v[e, pl.ds(16, 16)] = jnp.full((16,), mean * y, F32)

    @plsc.parallel_loop(0, n_edges, 1, unroll=4)
    def pass2(e):
        gv = scale_v[e, pl.ds(0, 16)]
        mg = scale_v[e, pl.ds(16, 16)]
        for j in range(nv):
            sl = pl.ds(j * 16, 16)
            u = buf_u[e, sl] * gv - mg
            u = u * lg[j] + lb[j]
            u = u / (1.0 + jnp.exp(-u))
            buf_u[e, sl] = u


def _make_edge_kernel(n_nodes, n_edges, hid, chunk):
    mesh = plsc.VectorSubcoreMesh(core_axis_name="c", subcore_axis_name="s")
    nc, ns = 2, 16
    epw = n_edges // (nc * ns)
    n_chunks = epw // chunk
    rpt = (n_nodes // ns) // 8 * 8
    tail = n_nodes - rpt * ns

    assert n_chunks % 2 == 0 and n_chunks >= 4

    @functools.partial(
        pl.kernel,
        mesh=mesh,
        out_type=jax.ShapeDtypeStruct((nc, n_nodes, hid), F32),
        scratch_types=[
            pltpu.VMEM((2, 2, chunk), jnp.int32),
            pltpu.VMEM((2, chunk), jnp.int32),
            pltpu.VMEM((2, chunk, hid), F32),
            pltpu.VMEM((2, chunk, hid), F32),
            pltpu.VMEM((2, chunk, hid // 2), jnp.int32),
            pltpu.VMEM((2, chunk, hid), F32),
            pltpu.VMEM((2, hid), F32),
            pltpu.VMEM((chunk, 32), F32),
            pltpu.VMEM_SHARED((n_nodes, hid), F32),
            pltpu.SemaphoreType.DMA,
            pltpu.SemaphoreType.DMA,
            pltpu.SemaphoreType.DMA,
            pltpu.SemaphoreType.DMA,
            pltpu.SemaphoreType.DMA,
            pltpu.SemaphoreType.DMA,
            pltpu.SemaphoreType.DMA,
            pltpu.SemaphoreType.DMA,
            pltpu.SemaphoreType.DMA,
            pltpu.SemaphoreType.DMA,
        ],
        compiler_params=pltpu.CompilerParams(needs_layout_passes=False),
    )
    def edge_k(a_hbm, b_hbm, c_hbm, rc_hbm, lglb_hbm, zero_hbm,
               out_hbm, rc_v, row_s, buf_a, buf_b, buf_c, buf_u, lglb_v,
               scale_v, s_sh, sem_i0, sem_i1, sem_a0, sem_a1, sem_b0,
               sem_b1, sem_c0, sem_c1, sem_u0, sem_u1):
        cid = lax.axis_index("c")
        sid = lax.axis_index("s")
        wid = cid * ns + sid
        ebase = wid * epw
        tbase = wid * n_chunks
        sem_i = (sem_i0, sem_i1)
        sem_a = (sem_a0, sem_a1)
        sem_b = (sem_b0, sem_b1)
        sem_c = (sem_c0, sem_c1)
        sem_u = (sem_u0, sem_u1)

        def start_idx(s, i):
            pltpu.async_copy(rc_hbm.at[tbase + i], rc_v.at[s], sem_i[s])

        def wait_idx(s, i):
            pltpu.make_async_copy(rc_hbm.at[tbase + i], rc_v.at[s],
                                  sem_i[s]).wait()

        def start_gather(s, i):
            off = ebase + i * chunk
            pltpu.async_copy(a_hbm.at[rc_v.at[s, 0]], buf_a.at[s], sem_a[s])
            pltpu.async_copy(b_hbm.at[rc_v.at[s, 1]], buf_b.at[s], sem_b[s])
            pltpu.async_copy(c_hbm.at[pl.ds(off, chunk)], buf_c.at[s],
                             sem_c[s])

        def wait_scatter(s):
            pltpu.make_async_copy(buf_u.at[s], s_sh.at[row_s.at[s]],
                                  sem_u[s]).wait()

        def finish(s, i):
            pltpu.make_async_copy(a_hbm.at[rc_v.at[s, 0]], buf_a.at[s],
                                  sem_a[s]).wait()
            pltpu.make_async_copy(b_hbm.at[rc_v.at[s, 1]], buf_b.at[s],
                                  sem_b[s]).wait()
            pltpu.make_async_copy(c_hbm.at[pl.ds(0, chunk)], buf_c.at[s],
                                  sem_c[s]).wait()

            @pl.when(i >= 2)
            def _():
                wait_scatter(s)
            offs = list(range(0, chunk - 15, 16))
            if chunk % 16:
                offs.append(chunk - 16)
            for o in offs:
                sl = pl.ds(o, 16)
                row_s[s, sl] = rc_v[s, 0, sl]
            _edge_compute(buf_a.at[s], buf_b.at[s], buf_c.at[s],
                          buf_u.at[s], lglb_v, scale_v, chunk, hid)
            pltpu.async_copy(buf_u.at[s], s_sh.at[row_s.at[s]], sem_u[s],
                             add=True)

        stripe = pl.ds(sid * rpt, rpt)
        pltpu.sync_copy(zero_hbm.at[stripe], s_sh.at[stripe])
        if tail:
            @pl.when(sid == ns - 1)
            def _():
                ts = pl.ds(ns * rpt, tail)
                pltpu.sync_copy(zero_hbm.at[ts], s_sh.at[ts])
        pltpu.sync_copy(lglb_hbm, lglb_v)
        plsc.subcore_barrier()

        start_idx(0, 0)
        start_idx(1, 1)
        wait_idx(0, 0)
        start_gather(0, 0)

        def pair_body(g, carry):
            for h in (0, 1):
                i = g * 2 + h
                s = h

                @pl.when(i + 1 < n_chunks)
                def _():
                    wait_idx(1 - s, i + 1)
                    start_gather(1 - s, i + 1)
                finish(s, i)

                @pl.when(i + 2 < n_chunks)
                def _():
                    start_idx(s, i + 2)
            return carry

        lax.fori_loop(0, n_chunks // 2, pair_body, 0)
        wait_scatter(0)
        wait_scatter(1)

        plsc.subcore_barrier()
        pltpu.sync_copy(s_sh.at[stripe], out_hbm.at[cid, stripe])
        if tail:
            @pl.when(sid == ns - 1)
            def _():
                ts = pl.ds(ns * rpt, tail)
                pltpu.sync_copy(s_sh.at[ts], out_hbm.at[cid, ts])

    return edge_k


def _make_deg_kernel(n_nodes, n_edges, chunk):
    mesh = plsc.VectorSubcoreMesh(core_axis_name="c", subcore_axis_name="s")
    nc, ns = 2, 16
    epw = n_edges // (nc * ns)
    kb = 8
    n_supers = epw // (kb * chunk)
    rpt = (n_nodes // ns) // 8 * 8
    tail = n_nodes - rpt * ns

    @functools.partial(
        pl.kernel,
        mesh=mesh,
        out_type=jax.ShapeDtypeStruct((nc, n_nodes, 16), F32),
        scratch_types=[
            pltpu.VMEM((kb, chunk), jnp.int32),
            pltpu.VMEM((chunk, 16), F32),
            pltpu.VMEM_SHARED((n_nodes, 16), F32),
            pltpu.SemaphoreType.DMA,
        ],
        compiler_params=pltpu.CompilerParams(needs_layout_passes=False),
    )
    def deg_k(row2_hbm, zero_hbm, out_hbm, idx_v, ones_v, d_sh, sem):
        cid = lax.axis_index("c")
        sid = lax.axis_index("s")
        tbase = (cid * ns + sid) * (epw // chunk)
        stripe = pl.ds(sid * rpt, rpt)
        pltpu.sync_copy(zero_hbm.at[stripe], d_sh.at[stripe])
        if tail:
            @pl.when(sid == ns - 1)
            def _():
                ts = pl.ds(ns * rpt, tail)
                pltpu.sync_copy(zero_hbm.at[ts], d_sh.at[ts])

        def fill(e, carry):
            ones_v[e, :] = jnp.full((16,), 1.0, F32)
            return carry

        lax.fori_loop(0, chunk, fill, 0)
        plsc.subcore_barrier()

        def super_body(t, carry):
            pltpu.sync_copy(row2_hbm.at[pl.ds(tbase + t * kb, kb)], idx_v)
            for j in range(kb):
                pltpu.async_copy(ones_v, d_sh.at[idx_v.at[j]], sem, add=True)
            for j in range(kb):
                pltpu.make_async_copy(ones_v, d_sh.at[idx_v.at[j]],
                                      sem).wait()
            return carry

        lax.fori_loop(0, n_supers, super_body, 0)
        plsc.subcore_barrier()
        pltpu.sync_copy(d_sh.at[stripe], out_hbm.at[cid, stripe])
        if tail:
            @pl.when(sid == ns - 1)
            def _():
                ts = pl.ds(ns * rpt, tail)
                pltpu.sync_copy(d_sh.at[ts], out_hbm.at[cid, ts])

    return deg_k


def kernel(h, edges, edge_attr, params):
    p = params
    n, d = h.shape
    e = edges.shape[1]
    hid = p["emb_in"]["W1"].shape[1]
    nl = len(p["layers"])
    row = edges[0]
    col = edges[1]

    def r1(v):
        return v.reshape(1, -1)

    bn = 2000
    chunk = 40
    bf16 = jnp.bfloat16

    perm = [j * 32 + 2 * t + h
            for j in range(hid // 32) for h in range(2) for t in range(16)]
    perm = jnp.array(perm, jnp.int32)

    zero_nh = jnp.zeros((n, hid), F32)
    zero_n16 = jnp.zeros((n, 16), F32)

    w1e_ev = jnp.stack([lyr["e_W1"][2 * hid:, 0::2] for lyr in p["layers"]])
    w1e_od = jnp.stack([lyr["e_W1"][2 * hid:, 1::2] for lyr in p["layers"]])
    b1_ev = jnp.stack([r1(lyr["e_b1"][0::2]) for lyr in p["layers"]])
    b1_od = jnp.stack([r1(lyr["e_b1"][1::2]) for lyr in p["layers"]])

    c_i32 = _tc_call(_c_body, e, 4000, (0,),
                     [edge_attr, w1e_ev, b1_ev, w1e_od, b1_od],
                     [jax.ShapeDtypeStruct((nl, e, hid // 2), jnp.int32)])

    deg_k = _make_deg_kernel(n, e, 125)
    deg16 = deg_k(row.reshape(e // 125, 125), zero_n16)

    lyr0 = p["layers"][0]
    xn, a_t, b_t = _tc_call(
        _in_body, n, bn, (0,),
        [h, p["emb_in"]["W1"], r1(p["emb_in"]["b1"]), p["emb_in"]["W2"],
         r1(p["emb_in"]["b2"]), r1(lyr0["ng"]), r1(lyr0["nb"]),
         lyr0["e_W1"][:hid][:, perm], lyr0["e_W1"][hid:2 * hid][:, perm]],
        [jax.ShapeDtypeStruct((n, hid), F32)] * 3)

    edge_k = _make_edge_kernel(n, e, hid, chunk)

    rc = jnp.stack([row.reshape(e // chunk, chunk),
                    col.reshape(e // chunk, chunk)], axis=1)
    for l in range(nl):
        lyr = p["layers"][l]
        lglb = jnp.stack([lyr["e_lg"], lyr["e_lb"]])[:, perm]
        s2 = edge_k(a_t, b_t, c_i32[l], rc, lglb, zero_nh)
        post = [s2, deg16, lyr["e_W2"][perm], r1(lyr["e_b2"]),
                lyr["n_W1"][:hid], lyr["n_W1"][hid:], r1(lyr["n_b1"]),
                r1(lyr["n_lg"]), r1(lyr["n_lb"]), lyr["n_W2"], r1(lyr["n_b2"])]
        if l < nl - 1:
            nxt = p["layers"][l + 1]
            xn, a_t, b_t = _tc_call(
                _mid_body, n, bn, (0, 1, 2),
                [xn] + post + [r1(nxt["ng"]), r1(nxt["nb"]),
                               nxt["e_W1"][:hid][:, perm],
                               nxt["e_W1"][hid:2 * hid][:, perm]],
                [jax.ShapeDtypeStruct((n, hid), F32)] * 3)
        else:
            out = _tc_call(
                _fin_body, n, bn, (0, 1, 2),
                [xn] + post + [p["emb_out"]["W1"], r1(p["emb_out"]["b1"]),
                               p["emb_out"]["W2"], r1(p["emb_out"]["b2"])],
                [jax.ShapeDtypeStruct((n, d), F32)])
    return out
